# baseline (speedup 1.0000x reference)
import functools

import jax
import jax.numpy as jnp
from jax import lax
from jax.experimental import pallas as pl
from jax.experimental.pallas import tpu as pltpu
from jax.experimental.pallas import tpu_sc as plsc

N = 10000
E = 320000
D = 128

NC = 2
NS = 16
NW = NC * NS
C = 128
EROWS = 2560
EPAD = EROWS * C - E
RPW = EROWS // NW
RPP = RPW // 2
RA = 80
RB = 80
PASSES = 2
SA = RA // PASSES
SB = RB // PASSES
SMAX = max(SA, SB, 2)
NP = N + 8
RPT = 624
TAIL = N - NS * RPT
TBASE = NS * RPT

_MESH = plsc.VectorSubcoreMesh(
    core_axis_name="c", subcore_axis_name="s", num_cores=NC, num_subcores=NS
)


def _seed_acc(cid, sid, zeros_hbm, acc):
    del cid
    rbase = sid * RPT
    pltpu.sync_copy(zeros_hbm.at[pl.ds(rbase, RPT)], acc.at[pl.ds(rbase, RPT)])

    @pl.when(sid == NS - 1)
    def _():
        pltpu.sync_copy(zeros_hbm.at[pl.ds(TBASE, TAIL)], acc.at[pl.ds(TBASE, TAIL)])


def _copy_out(cid, sid, acc, out_hbm):
    rbase = sid * RPT
    pltpu.sync_copy(acc.at[pl.ds(rbase, RPT)], out_hbm.at[cid, pl.ds(rbase, RPT)])

    @pl.when(sid == NS - 1)
    def _():
        pltpu.sync_copy(acc.at[pl.ds(TBASE, TAIL)], out_hbm.at[cid, pl.ds(TBASE, TAIL)])


@functools.partial(
    pl.kernel,
    out_type=jax.ShapeDtypeStruct((NC, N, D), jnp.float32),
    mesh=_MESH,
    scratch_types=[
        pltpu.VMEM((SMAX, C), jnp.int32),
        pltpu.VMEM((SMAX, C), jnp.int32),
        pltpu.VMEM((C, D), jnp.float32),
        pltpu.VMEM((C, D), jnp.float32),
        pltpu.VMEM_SHARED((NP, D), jnp.float32),
        pltpu.SemaphoreType.DMA,
        pltpu.SemaphoreType.DMA,
    ],
)
def _sc_aggregate(ha_hbm, hb_hbm, zeros_hbm, src_hbm, dst_hbm, out_hbm,
                  src_t, dst_t, rows0, rows1, acc, sem0, sem1):
    cid = lax.axis_index("c")
    sid = lax.axis_index("s")

    _seed_acc(cid, sid, zeros_hbm, acc)
    plsc.subcore_barrier()

    def run(h_hbm, row_base, span):
        pltpu.sync_copy(src_hbm.at[pl.ds(row_base, span)],
                        src_t.at[pl.ds(0, span)])
        pltpu.sync_copy(dst_hbm.at[pl.ds(row_base, span)],
                        dst_t.at[pl.ds(0, span)])
        pltpu.async_copy(h_hbm.at[src_t.at[0]], rows0, sem0)

        def pair(p, carry):
            i0 = 2 * p
            pltpu.async_copy(h_hbm.at[src_t.at[i0 + 1]], rows1, sem1)
            pltpu.make_async_copy(h_hbm.at[src_t.at[i0]], rows0, sem0).wait()
            pltpu.sync_copy(rows0, acc.at[dst_t.at[i0]], add=True)

            @pl.when(i0 + 2 < span)
            def _():
                pltpu.async_copy(h_hbm.at[src_t.at[i0 + 2]], rows0, sem0)

            pltpu.make_async_copy(h_hbm.at[src_t.at[i0 + 1]], rows1, sem1).wait()
            pltpu.sync_copy(rows1, acc.at[dst_t.at[i0 + 1]], add=True)
            return carry

        lax.fori_loop(0, span // 2, pair, 0)

    @pl.when(cid == 0)
    def _():
        for half in range(PASSES):
            run(ha_hbm, sid * RA + half * SA, SA)

    if RB:
        @pl.when(cid != 0)
        def _():
            for half in range(PASSES):
                run(hb_hbm, NS * RA + sid * RB + half * SB, SB)

    plsc.subcore_barrier()
    _copy_out(cid, sid, acc, out_hbm)


@functools.partial(
    pl.kernel,
    out_type=jax.ShapeDtypeStruct((NC, N, D), jnp.float32),
    mesh=_MESH,
    scratch_types=[
        pltpu.VMEM((RPW, C), jnp.int32),
        pltpu.VMEM((C, D), jnp.float32),
        pltpu.VMEM_SHARED((NP, D), jnp.float32),
    ],
)
def _sc_degree(dst_hbm, zeros_hbm, ones_hbm, out_hbm, dst_t, ones_t, acc):
    cid = lax.axis_index("c")
    sid = lax.axis_index("s")
    wid = sid * NC + cid

    _seed_acc(cid, sid, zeros_hbm, acc)
    pltpu.sync_copy(dst_hbm.at[pl.ds(wid * RPW, RPW)], dst_t)
    pltpu.sync_copy(ones_hbm, ones_t)
    plsc.subcore_barrier()

    def chunk(i, carry):
        pltpu.sync_copy(ones_t, acc.at[dst_t.at[i]], add=True)
        return carry

    lax.fori_loop(0, RPW, chunk, 0)
    plsc.subcore_barrier()
    _copy_out(cid, sid, acc, out_hbm)


BR = 400
GRID = N // BR


def _dinv_block(degp):
    deg = degp[0, :, :1] + degp[1, :, :1] + 1.0
    return lax.rsqrt(deg)


def _tc1_body(x_ref, w_ref, degp_ref, out_ref, outb_ref):
    dinv = _dinv_block(degp_ref[...])
    h = jnp.dot(x_ref[...], w_ref[...], preferred_element_type=jnp.float32)
    hp = h * dinv
    out_ref[...] = hp
    outb_ref[...] = hp


@jax.jit
def _tc1(x, W1, degp):
    return pl.pallas_call(
        _tc1_body,
        grid=(GRID,),
        in_specs=[
            pl.BlockSpec((BR, D), lambda i: (i, 0)),
            pl.BlockSpec((D, D), lambda i: (0, 0)),
            pl.BlockSpec((NC, BR, D), lambda i: (0, i, 0)),
        ],
        out_specs=[pl.BlockSpec((BR, D), lambda i: (i, 0)),
                   pl.BlockSpec((BR, D), lambda i: (i, 0))],
        out_shape=[jax.ShapeDtypeStruct((N, D), jnp.float32),
                   jax.ShapeDtypeStruct((N, D), jnp.float32)],
    )(x, W1, degp)


def _tc2_body(part_ref, h_ref, degp_ref, b_ref, w_ref, out_ref, outb_ref):
    dinv = _dinv_block(degp_ref[...])
    s = part_ref[0] + part_ref[1] + h_ref[...]
    t = jnp.maximum(s * dinv + b_ref[...], 0.0)
    h = jnp.dot(t, w_ref[...], preferred_element_type=jnp.float32)
    hp = h * dinv
    out_ref[...] = hp
    outb_ref[...] = hp


@jax.jit
def _tc2(part, h1, degp, b1, W2):
    return pl.pallas_call(
        _tc2_body,
        grid=(GRID,),
        in_specs=[
            pl.BlockSpec((NC, BR, D), lambda i: (0, i, 0)),
            pl.BlockSpec((BR, D), lambda i: (i, 0)),
            pl.BlockSpec((NC, BR, D), lambda i: (0, i, 0)),
            pl.BlockSpec((D,), lambda i: (0,)),
            pl.BlockSpec((D, D), lambda i: (0, 0)),
        ],
        out_specs=[pl.BlockSpec((BR, D), lambda i: (i, 0)),
                   pl.BlockSpec((BR, D), lambda i: (i, 0))],
        out_shape=[jax.ShapeDtypeStruct((N, D), jnp.float32),
                   jax.ShapeDtypeStruct((N, D), jnp.float32)],
    )(part, h1, degp, b1, W2)


def _tc3_body(part_ref, h_ref, degp_ref, b_ref, out_ref):
    dinv = _dinv_block(degp_ref[...])
    s = part_ref[0] + part_ref[1] + h_ref[...]
    out_ref[...] = s * dinv + b_ref[...]


@jax.jit
def _tc3(part, h2, degp, b2):
    return pl.pallas_call(
        _tc3_body,
        grid=(GRID,),
        in_specs=[
            pl.BlockSpec((NC, BR, D), lambda i: (0, i, 0)),
            pl.BlockSpec((BR, D), lambda i: (i, 0)),
            pl.BlockSpec((NC, BR, D), lambda i: (0, i, 0)),
            pl.BlockSpec((D,), lambda i: (0,)),
        ],
        out_specs=pl.BlockSpec((BR, D), lambda i: (i, 0)),
        out_shape=jax.ShapeDtypeStruct((N, D), jnp.float32),
    )(part, h2, degp, b2)


def kernel(x, edge_index, W1, b1, W2, b2):
    src = edge_index[0]
    dst = edge_index[1]
    pad_src = jnp.zeros((EPAD,), jnp.int32)
    pad_dst = jnp.full((EPAD,), N, jnp.int32)
    src2d = jnp.concatenate([src, pad_src]).reshape(EROWS, C)
    dst2d = jnp.concatenate([dst, pad_dst]).reshape(EROWS, C)

    zeros_nd = jnp.zeros((N, D), jnp.float32)
    ones_cd = jnp.ones((C, D), jnp.float32)

    degp = _sc_degree(dst2d, zeros_nd, ones_cd)
    h1a, h1b = _tc1(x, W1, degp)
    p1 = _sc_aggregate(h1a, h1b, zeros_nd, src2d, dst2d)
    h2a, h2b = _tc2(p1, h1a, degp, b1, W2)
    p2 = _sc_aggregate(h2a, h2b, zeros_nd, src2d, dst2d)
    return _tc3(p2, h2a, degp, b2)


# speedup vs baseline: 1.2074x; 1.2074x over previous
import functools

import jax
import jax.numpy as jnp
from jax import lax
from jax.experimental import pallas as pl
from jax.experimental.pallas import tpu as pltpu
from jax.experimental.pallas import tpu_sc as plsc

N = 10000
E = 320000
D = 128

NC = 2
NS = 16
NW = NC * NS
C = 128
EROWS = 2560
EPAD = EROWS * C - E
RPW = EROWS // NW
RPP = RPW // 2
RA = 112
RB = 48
PASSES = 2
SA = RA // PASSES
SB = RB // PASSES
SMAX = max(SA, SB, 2)
NP = N + 8
RPT = 624
TAIL = N - NS * RPT
TBASE = NS * RPT

_MESH = plsc.VectorSubcoreMesh(
    core_axis_name="c", subcore_axis_name="s", num_cores=NC, num_subcores=NS
)


def _seed_acc(cid, sid, zeros_hbm, acc):
    del cid
    rbase = sid * RPT
    pltpu.sync_copy(zeros_hbm.at[pl.ds(rbase, RPT)], acc.at[pl.ds(rbase, RPT)])

    @pl.when(sid == NS - 1)
    def _():
        pltpu.sync_copy(zeros_hbm.at[pl.ds(TBASE, TAIL)], acc.at[pl.ds(TBASE, TAIL)])


def _copy_out(cid, sid, acc, out_hbm):
    rbase = sid * RPT
    pltpu.sync_copy(acc.at[pl.ds(rbase, RPT)], out_hbm.at[cid, pl.ds(rbase, RPT)])

    @pl.when(sid == NS - 1)
    def _():
        pltpu.sync_copy(acc.at[pl.ds(TBASE, TAIL)], out_hbm.at[cid, pl.ds(TBASE, TAIL)])


@functools.partial(
    pl.kernel,
    out_type=jax.ShapeDtypeStruct((NC, N, D), jnp.float32),
    mesh=_MESH,
    scratch_types=[
        pltpu.VMEM((SMAX, C), jnp.int32),
        pltpu.VMEM((SMAX, C), jnp.int32),
        pltpu.VMEM((C, D), jnp.float32),
        pltpu.VMEM((C, D), jnp.float32),
        pltpu.VMEM_SHARED((NP, D), jnp.float32),
        pltpu.SemaphoreType.DMA,
        pltpu.SemaphoreType.DMA,
    ],
)
def _sc_aggregate(h_hbm, zeros_hbm, src_hbm, dst_hbm, out_hbm,
                  src_t, dst_t, rows0, rows1, acc, sem0, sem1):
    cid = lax.axis_index("c")
    sid = lax.axis_index("s")

    _seed_acc(cid, sid, zeros_hbm, acc)
    plsc.subcore_barrier()

    def run(row_base, span):
        pltpu.sync_copy(src_hbm.at[pl.ds(row_base, span)],
                        src_t.at[pl.ds(0, span)])
        pltpu.sync_copy(dst_hbm.at[pl.ds(row_base, span)],
                        dst_t.at[pl.ds(0, span)])
        pltpu.async_copy(h_hbm.at[src_t.at[0]], rows0, sem0)

        def pair(p, carry):
            i0 = 2 * p
            pltpu.async_copy(h_hbm.at[src_t.at[i0 + 1]], rows1, sem1)
            pltpu.make_async_copy(h_hbm.at[src_t.at[i0]], rows0, sem0).wait()
            pltpu.sync_copy(rows0, acc.at[dst_t.at[i0]], add=True)

            @pl.when(i0 + 2 < span)
            def _():
                pltpu.async_copy(h_hbm.at[src_t.at[i0 + 2]], rows0, sem0)

            pltpu.make_async_copy(h_hbm.at[src_t.at[i0 + 1]], rows1, sem1).wait()
            pltpu.sync_copy(rows1, acc.at[dst_t.at[i0 + 1]], add=True)
            return carry

        lax.fori_loop(0, span // 2, pair, 0)

    @pl.when(cid == 0)
    def _():
        for half in range(PASSES):
            run(sid * RA + half * SA, SA)

    if RB:
        @pl.when(cid != 0)
        def _():
            for half in range(PASSES):
                run(NS * RA + sid * RB + half * SB, SB)

    plsc.subcore_barrier()
    _copy_out(cid, sid, acc, out_hbm)


@functools.partial(
    pl.kernel,
    out_type=jax.ShapeDtypeStruct((NC, N, D), jnp.float32),
    mesh=_MESH,
    scratch_types=[
        pltpu.VMEM((RPW, C), jnp.int32),
        pltpu.VMEM((C, D), jnp.float32),
        pltpu.VMEM_SHARED((NP, D), jnp.float32),
    ],
)
def _sc_degree(dst_hbm, zeros_hbm, ones_hbm, out_hbm, dst_t, ones_t, acc):
    cid = lax.axis_index("c")
    sid = lax.axis_index("s")
    wid = sid * NC + cid

    _seed_acc(cid, sid, zeros_hbm, acc)
    pltpu.sync_copy(dst_hbm.at[pl.ds(wid * RPW, RPW)], dst_t)
    pltpu.sync_copy(ones_hbm, ones_t)
    plsc.subcore_barrier()

    def chunk(i, carry):
        pltpu.sync_copy(ones_t, acc.at[dst_t.at[i]], add=True)
        return carry

    lax.fori_loop(0, RPW, chunk, 0)
    plsc.subcore_barrier()
    _copy_out(cid, sid, acc, out_hbm)


BR = 400
GRID = N // BR


def _dinv_block(degp):
    deg = degp[0, :, :1] + degp[1, :, :1] + 1.0
    return lax.rsqrt(deg)


def _tc1_body(x_ref, w_ref, degp_ref, out_ref):
    dinv = _dinv_block(degp_ref[...])
    h = jnp.dot(x_ref[...], w_ref[...], preferred_element_type=jnp.float32)
    out_ref[...] = h * dinv


@jax.jit
def _tc1(x, W1, degp):
    return pl.pallas_call(
        _tc1_body,
        grid=(GRID,),
        in_specs=[
            pl.BlockSpec((BR, D), lambda i: (i, 0)),
            pl.BlockSpec((D, D), lambda i: (0, 0)),
            pl.BlockSpec((NC, BR, D), lambda i: (0, i, 0)),
        ],
        out_specs=pl.BlockSpec((BR, D), lambda i: (i, 0)),
        out_shape=jax.ShapeDtypeStruct((N, D), jnp.float32),
    )(x, W1, degp)


def _tc2_body(part_ref, h_ref, degp_ref, b_ref, w_ref, out_ref):
    dinv = _dinv_block(degp_ref[...])
    s = part_ref[0] + part_ref[1] + h_ref[...]
    t = jnp.maximum(s * dinv + b_ref[...], 0.0)
    h = jnp.dot(t, w_ref[...], preferred_element_type=jnp.float32)
    out_ref[...] = h * dinv


@jax.jit
def _tc2(part, h1, degp, b1, W2):
    return pl.pallas_call(
        _tc2_body,
        grid=(GRID,),
        in_specs=[
            pl.BlockSpec((NC, BR, D), lambda i: (0, i, 0)),
            pl.BlockSpec((BR, D), lambda i: (i, 0)),
            pl.BlockSpec((NC, BR, D), lambda i: (0, i, 0)),
            pl.BlockSpec((D,), lambda i: (0,)),
            pl.BlockSpec((D, D), lambda i: (0, 0)),
        ],
        out_specs=pl.BlockSpec((BR, D), lambda i: (i, 0)),
        out_shape=jax.ShapeDtypeStruct((N, D), jnp.float32),
    )(part, h1, degp, b1, W2)


def _tc3_body(part_ref, h_ref, degp_ref, b_ref, out_ref):
    dinv = _dinv_block(degp_ref[...])
    s = part_ref[0] + part_ref[1] + h_ref[...]
    out_ref[...] = s * dinv + b_ref[...]


@jax.jit
def _tc3(part, h2, degp, b2):
    return pl.pallas_call(
        _tc3_body,
        grid=(GRID,),
        in_specs=[
            pl.BlockSpec((NC, BR, D), lambda i: (0, i, 0)),
            pl.BlockSpec((BR, D), lambda i: (i, 0)),
            pl.BlockSpec((NC, BR, D), lambda i: (0, i, 0)),
            pl.BlockSpec((D,), lambda i: (0,)),
        ],
        out_specs=pl.BlockSpec((BR, D), lambda i: (i, 0)),
        out_shape=jax.ShapeDtypeStruct((N, D), jnp.float32),
    )(part, h2, degp, b2)


def kernel(x, edge_index, W1, b1, W2, b2):
    src = edge_index[0]
    dst = edge_index[1]
    pad_src = jnp.zeros((EPAD,), jnp.int32)
    pad_dst = jnp.full((EPAD,), N, jnp.int32)
    src2d = jnp.concatenate([src, pad_src]).reshape(EROWS, C)
    dst2d = jnp.concatenate([dst, pad_dst]).reshape(EROWS, C)

    zeros_nd = jnp.zeros((N, D), jnp.float32)
    ones_cd = jnp.ones((C, D), jnp.float32)

    degp = _sc_degree(dst2d, zeros_nd, ones_cd)
    h1 = _tc1(x, W1, degp)
    p1 = _sc_aggregate(h1, zeros_nd, src2d, dst2d)
    h2 = _tc2(p1, h1, degp, b1, W2)
    p2 = _sc_aggregate(h2, zeros_nd, src2d, dst2d)
    return _tc3(p2, h2, degp, b2)
